# Initial kernel scaffold; baseline (speedup 1.0000x reference)
import functools

import jax
import jax.numpy as jnp
from jax import lax
from jax.experimental import pallas as pl
from jax.experimental.pallas import tpu as pltpu
from jax.experimental.pallas import tpu_sc as plsc

N = 10000
E = 320000
D_IN = 128
D_HID = 128
N_CLASSES = 64

NC = 2
NS = 16
NW = NC * NS
CH = 128
K = 80
E_PAD = NW * K * CH
NACC = N + 16
GZ = NACC // NS
RPS = N // NS

D1P = D_IN + 16
D2P = N_CLASSES

BR = 400


def _sc_agg_body(dp, g_hbm, src_hbm, dst_hbm, zeros_hbm, out_a, out_b,
                 src_v, dst_v, buf0, buf1, acc, sem0, sem1):
    cid = lax.axis_index("c")
    sid = lax.axis_index("s")
    wid = sid * NC + cid

    pltpu.sync_copy(src_hbm.at[wid], src_v)
    pltpu.sync_copy(dst_hbm.at[wid], dst_v)
    pltpu.sync_copy(zeros_hbm.at[pl.ds(sid * GZ, GZ)], acc.at[pl.ds(sid * GZ, GZ)])
    pltpu.async_copy(g_hbm.at[src_v.at[0]], buf0, sem0)
    plsc.subcore_barrier()

    @pl.loop(0, K, step=2)
    def _(j):
        pltpu.async_copy(g_hbm.at[src_v.at[j + 1]], buf1, sem1)
        pltpu.make_async_copy(g_hbm.at[src_v.at[j]], buf0, sem0).wait()
        pltpu.sync_copy(buf0, acc.at[dst_v.at[j]], add=True)

        @pl.when(j + 2 < K)
        def _():
            pltpu.async_copy(g_hbm.at[src_v.at[j + 2]], buf0, sem0)

        pltpu.make_async_copy(g_hbm.at[src_v.at[j + 1]], buf1, sem1).wait()
        pltpu.sync_copy(buf1, acc.at[dst_v.at[j + 1]], add=True)

    plsc.subcore_barrier()

    @pl.when(cid == 0)
    def _():
        pltpu.sync_copy(acc.at[pl.ds(sid * RPS, RPS)], out_a.at[pl.ds(sid * RPS, RPS)])

    @pl.when(cid == 1)
    def _():
        pltpu.sync_copy(acc.at[pl.ds(sid * RPS, RPS)], out_b.at[pl.ds(sid * RPS, RPS)])


def _sc_aggregate(dp, g, src_r, dst_r, zeros):
    mesh = plsc.VectorSubcoreMesh(core_axis_name="c", subcore_axis_name="s")
    part = jax.ShapeDtypeStruct((N, dp), jnp.float32)
    kern = pl.kernel(
        functools.partial(_sc_agg_body, dp),
        out_type=(part, part),
        mesh=mesh,
        scratch_types=[
            pltpu.VMEM((K, CH), jnp.int32),
            pltpu.VMEM((K, CH), jnp.int32),
            pltpu.VMEM((CH, dp), jnp.float32),
            pltpu.VMEM((CH, dp), jnp.float32),
            pltpu.VMEM_SHARED((NACC, dp), jnp.float32),
            pltpu.SemaphoreType.DMA,
            pltpu.SemaphoreType.DMA,
        ],
        name=f"sc_segsum_d{dp}",
    )
    return kern(g, src_r, dst_r, zeros)


def _mm_aug_kernel(x_ref, w_ref, o_ref):
    y = jnp.dot(x_ref[...], w_ref[...], preferred_element_type=jnp.float32)
    col = lax.broadcasted_iota(jnp.int32, (BR, D1P), 1)
    o_ref[...] = y + jnp.where(col == D_IN, 1.0, 0.0).astype(jnp.float32)


def _stage2_kernel(sa_ref, sb_ref, g1_ref, b1_ref, w2_ref, g2_ref, inv_ref):
    sa = sa_ref[...]
    sb = sb_ref[...]
    msg = sa[:, :D_IN] + sb[:, :D_IN]
    deg = sa[:, D_IN:D_IN + 1] + sb[:, D_IN:D_IN + 1]
    inv = 1.0 / (deg + 1.0)
    h = jnp.maximum((msg + g1_ref[:, :D_IN]) * inv + b1_ref[...], 0.0)
    g2_ref[...] = jnp.dot(h, w2_ref[...], preferred_element_type=jnp.float32)
    inv_ref[...] = inv


def _stage3_kernel(sa_ref, sb_ref, g2_ref, inv_ref, b2_ref, o_ref):
    o_ref[...] = ((sa_ref[...] + sb_ref[...] + g2_ref[...]) * inv_ref[...]
                  + b2_ref[...])


def kernel(features, edge_index, W1, b1, W2, b2):
    src = edge_index[0]
    dst = edge_index[1]
    pad = E_PAD - E
    src_r = jnp.concatenate([src, jnp.zeros((pad,), jnp.int32)]).reshape(NW, K, CH)
    dst_r = jnp.concatenate([dst, jnp.full((pad,), N, jnp.int32)]).reshape(NW, K, CH)
    zeros1 = jnp.zeros((NACC, D1P), jnp.float32)
    zeros2 = jnp.zeros((NACC, D2P), jnp.float32)
    w1p = jnp.pad(W1, ((0, 0), (0, D1P - D_HID)))
    b1r = b1.reshape(1, D_HID)
    b2r = b2.reshape(1, N_CLASSES)

    grid = N // BR

    g1 = pl.pallas_call(
        _mm_aug_kernel,
        grid=(grid,),
        in_specs=[
            pl.BlockSpec((BR, D_IN), lambda i: (i, 0)),
            pl.BlockSpec((D_IN, D1P), lambda i: (0, 0)),
        ],
        out_specs=pl.BlockSpec((BR, D1P), lambda i: (i, 0)),
        out_shape=jax.ShapeDtypeStruct((N, D1P), jnp.float32),
    )(features, w1p)

    sa1, sb1 = _sc_aggregate(D1P, g1, src_r, dst_r, zeros1)

    g2, inv = pl.pallas_call(
        _stage2_kernel,
        grid=(grid,),
        in_specs=[
            pl.BlockSpec((BR, D1P), lambda i: (i, 0)),
            pl.BlockSpec((BR, D1P), lambda i: (i, 0)),
            pl.BlockSpec((BR, D1P), lambda i: (i, 0)),
            pl.BlockSpec((1, D_HID), lambda i: (0, 0)),
            pl.BlockSpec((D_HID, N_CLASSES), lambda i: (0, 0)),
        ],
        out_specs=[
            pl.BlockSpec((BR, N_CLASSES), lambda i: (i, 0)),
            pl.BlockSpec((BR, 1), lambda i: (i, 0)),
        ],
        out_shape=[
            jax.ShapeDtypeStruct((N, N_CLASSES), jnp.float32),
            jax.ShapeDtypeStruct((N, 1), jnp.float32),
        ],
    )(sa1, sb1, g1, b1r, W2)

    sa2, sb2 = _sc_aggregate(D2P, g2, src_r, dst_r, zeros2)

    out = pl.pallas_call(
        _stage3_kernel,
        grid=(grid,),
        in_specs=[
            pl.BlockSpec((BR, N_CLASSES), lambda i: (i, 0)),
            pl.BlockSpec((BR, N_CLASSES), lambda i: (i, 0)),
            pl.BlockSpec((BR, N_CLASSES), lambda i: (i, 0)),
            pl.BlockSpec((BR, 1), lambda i: (i, 0)),
            pl.BlockSpec((1, N_CLASSES), lambda i: (0, 0)),
        ],
        out_specs=pl.BlockSpec((BR, N_CLASSES), lambda i: (i, 0)),
        out_shape=jax.ShapeDtypeStruct((N, N_CLASSES), jnp.float32),
    )(sa2, sb2, g2, inv, b2r)

    return out


# speedup vs baseline: 3.6426x; 3.6426x over previous
import functools

import jax
import jax.numpy as jnp
from jax import lax
from jax.experimental import pallas as pl
from jax.experimental.pallas import tpu as pltpu
from jax.experimental.pallas import tpu_sc as plsc

N = 10000
E = 320000
D_IN = 128
D_HID = 128
N_CLASSES = 64
DH = 64

NC = 2
NS = 16
NW = NC * NS
CH = 128
K = 80
E_PAD = NW * K * CH
NACC = 10112
GZ = NACC // NS

DPA = 80
DPB = 64
DP2 = N_CLASSES

BR = 400


def _sc_agg_body(dp, g_hbm, src_hbm, dst_hbm, zeros_hbm, out_a, out_b,
                 src_v, dst_v, buf0, buf1, acc, sem0, sem1):
    cid = lax.axis_index("c")
    sid = lax.axis_index("s")
    wid = sid * NC + cid

    pltpu.sync_copy(src_hbm.at[wid], src_v)
    pltpu.sync_copy(dst_hbm.at[wid], dst_v)
    pltpu.sync_copy(zeros_hbm.at[pl.ds(sid * GZ, GZ)], acc.at[pl.ds(sid * GZ, GZ)])
    pltpu.async_copy(g_hbm.at[src_v.at[0]], buf0, sem0)
    plsc.subcore_barrier()

    @pl.loop(0, K, step=2)
    def _(j):
        pltpu.async_copy(g_hbm.at[src_v.at[j + 1]], buf1, sem1)
        pltpu.make_async_copy(g_hbm.at[src_v.at[j]], buf0, sem0).wait()
        pltpu.sync_copy(buf0, acc.at[dst_v.at[j]], add=True)

        @pl.when(j + 2 < K)
        def _():
            pltpu.async_copy(g_hbm.at[src_v.at[j + 2]], buf0, sem0)

        pltpu.make_async_copy(g_hbm.at[src_v.at[j + 1]], buf1, sem1).wait()
        pltpu.sync_copy(buf1, acc.at[dst_v.at[j + 1]], add=True)

    plsc.subcore_barrier()

    @pl.when(cid == 0)
    def _():
        pltpu.sync_copy(acc.at[pl.ds(sid * GZ, GZ)], out_a.at[pl.ds(sid * GZ, GZ)])

    @pl.when(cid == 1)
    def _():
        pltpu.sync_copy(acc.at[pl.ds(sid * GZ, GZ)], out_b.at[pl.ds(sid * GZ, GZ)])


def _sc_aggregate(dp, g, src_r, dst_r, zeros):
    mesh = plsc.VectorSubcoreMesh(core_axis_name="c", subcore_axis_name="s")
    part = jax.ShapeDtypeStruct((NACC, dp), jnp.float32)
    kern = pl.kernel(
        functools.partial(_sc_agg_body, dp),
        out_type=(part, part),
        mesh=mesh,
        scratch_types=[
            pltpu.VMEM((K, CH), jnp.int32),
            pltpu.VMEM((K, CH), jnp.int32),
            pltpu.VMEM((CH, dp), jnp.float32),
            pltpu.VMEM((CH, dp), jnp.float32),
            pltpu.VMEM_SHARED((NACC, dp), jnp.float32),
            pltpu.SemaphoreType.DMA,
            pltpu.SemaphoreType.DMA,
        ],
        name=f"sc_segsum_d{dp}",
        compiler_params=pltpu.CompilerParams(use_tc_tiling_on_sc=False),
    )
    return kern(g, src_r, dst_r, zeros)


def _mm_aug_kernel(x_ref, wa_ref, wb_ref, oa_ref, ob_ref):
    x = x_ref[...]
    ya = jnp.dot(x, wa_ref[...], preferred_element_type=jnp.float32)
    col = lax.broadcasted_iota(jnp.int32, (BR, DPA), 1)
    oa_ref[...] = ya + jnp.where(col == DH, 1.0, 0.0).astype(jnp.float32)
    ob_ref[...] = jnp.dot(x, wb_ref[...], preferred_element_type=jnp.float32)


def _stage2_kernel(saa_ref, sba_ref, sab_ref, sbb_ref, g1a_ref, g1b_ref,
                   b1_ref, w2_ref, g2_ref, inv_ref):
    ma = saa_ref[...] + sba_ref[...]
    mb = sab_ref[...] + sbb_ref[...]
    deg = ma[:, DH:DH + 1]
    inv = 1.0 / (deg + 1.0)
    agg = jnp.concatenate(
        [ma[:, :DH] + g1a_ref[:, :DH], mb + g1b_ref[...]], axis=1)
    h = jnp.maximum(agg * inv + b1_ref[...], 0.0)
    g2_ref[...] = jnp.dot(h, w2_ref[...], preferred_element_type=jnp.float32)
    inv_ref[...] = inv


def _stage3_kernel(sa_ref, sb_ref, g2_ref, inv_ref, b2_ref, o_ref):
    o_ref[...] = ((sa_ref[...] + sb_ref[...] + g2_ref[...]) * inv_ref[...]
                  + b2_ref[...])


def kernel(features, edge_index, W1, b1, W2, b2):
    src = edge_index[0]
    dst = edge_index[1]
    pad = E_PAD - E
    src_r = jnp.concatenate([src, jnp.zeros((pad,), jnp.int32)]).reshape(NW, K, CH)
    dst_r = jnp.concatenate([dst, jnp.full((pad,), N, jnp.int32)]).reshape(NW, K, CH)
    zeros_a = jnp.zeros((NACC, DPA), jnp.float32)
    zeros_b = jnp.zeros((NACC, DPB), jnp.float32)
    w1a = jnp.pad(W1[:, :DH], ((0, 0), (0, DPA - DH)))
    w1b = W1[:, DH:]
    b1r = b1.reshape(1, D_HID)
    b2r = b2.reshape(1, N_CLASSES)

    grid = N // BR

    g1a, g1b = pl.pallas_call(
        _mm_aug_kernel,
        grid=(grid,),
        in_specs=[
            pl.BlockSpec((BR, D_IN), lambda i: (i, 0)),
            pl.BlockSpec((D_IN, DPA), lambda i: (0, 0)),
            pl.BlockSpec((D_IN, DPB), lambda i: (0, 0)),
        ],
        out_specs=[
            pl.BlockSpec((BR, DPA), lambda i: (i, 0)),
            pl.BlockSpec((BR, DPB), lambda i: (i, 0)),
        ],
        out_shape=[
            jax.ShapeDtypeStruct((N, DPA), jnp.float32),
            jax.ShapeDtypeStruct((N, DPB), jnp.float32),
        ],
    )(features, w1a, w1b)

    saa, sba = _sc_aggregate(DPA, g1a, src_r, dst_r, zeros_a)
    sab, sbb = _sc_aggregate(DPB, g1b, src_r, dst_r, zeros_b)

    g2, inv = pl.pallas_call(
        _stage2_kernel,
        grid=(grid,),
        in_specs=[
            pl.BlockSpec((BR, DPA), lambda i: (i, 0)),
            pl.BlockSpec((BR, DPA), lambda i: (i, 0)),
            pl.BlockSpec((BR, DPB), lambda i: (i, 0)),
            pl.BlockSpec((BR, DPB), lambda i: (i, 0)),
            pl.BlockSpec((BR, DPA), lambda i: (i, 0)),
            pl.BlockSpec((BR, DPB), lambda i: (i, 0)),
            pl.BlockSpec((1, D_HID), lambda i: (0, 0)),
            pl.BlockSpec((D_HID, N_CLASSES), lambda i: (0, 0)),
        ],
        out_specs=[
            pl.BlockSpec((BR, N_CLASSES), lambda i: (i, 0)),
            pl.BlockSpec((BR, 1), lambda i: (i, 0)),
        ],
        out_shape=[
            jax.ShapeDtypeStruct((N, N_CLASSES), jnp.float32),
            jax.ShapeDtypeStruct((N, 1), jnp.float32),
        ],
    )(saa, sba, sab, sbb, g1a, g1b, b1r, W2)

    sa2, sb2 = _sc_aggregate(DP2, g2, src_r, dst_r, zeros_b)

    out = pl.pallas_call(
        _stage3_kernel,
        grid=(grid,),
        in_specs=[
            pl.BlockSpec((BR, N_CLASSES), lambda i: (i, 0)),
            pl.BlockSpec((BR, N_CLASSES), lambda i: (i, 0)),
            pl.BlockSpec((BR, N_CLASSES), lambda i: (i, 0)),
            pl.BlockSpec((BR, 1), lambda i: (i, 0)),
            pl.BlockSpec((1, N_CLASSES), lambda i: (0, 0)),
        ],
        out_specs=pl.BlockSpec((BR, N_CLASSES), lambda i: (i, 0)),
        out_shape=jax.ShapeDtypeStruct((N, N_CLASSES), jnp.float32),
    )(sa2, sb2, g2, inv, b2r)

    return out


# speedup vs baseline: 3.6660x; 1.0064x over previous
import functools

import jax
import jax.numpy as jnp
from jax import lax
from jax.experimental import pallas as pl
from jax.experimental.pallas import tpu as pltpu
from jax.experimental.pallas import tpu_sc as plsc

N = 10000
E = 320000
D_IN = 128
D_HID = 128
N_CLASSES = 64
DH = 64

NC = 2
NS = 16
NW = NC * NS
CH = 128
K = 80
E_PAD = NW * K * CH
NACC = 10112
GZ = NACC // NS

DPA = 80
DPB = 64
DP2 = N_CLASSES

BR = 400


NBUF = 4


def _sc_agg_body(dp, g_hbm, src_hbm, dst_hbm, zeros_hbm, out_a, out_b,
                 src_v, dst_v, b0, b1, b2, b3, acc,
                 zsem, g0, g1, g2, g3, s0, s1, s2, s3):
    bufs = (b0, b1, b2, b3)
    gsems = (g0, g1, g2, g3)
    ssems = (s0, s1, s2, s3)
    cid = lax.axis_index("c")
    sid = lax.axis_index("s")
    wid = sid * NC + cid
    rows = pl.ds(sid * GZ, GZ)

    pltpu.async_copy(zeros_hbm.at[rows], acc.at[rows], zsem)
    pltpu.sync_copy(src_hbm.at[wid], src_v)
    pltpu.sync_copy(dst_hbm.at[wid], dst_v)
    for b in range(NBUF):
        pltpu.async_copy(g_hbm.at[src_v.at[b]], bufs[b], gsems[b])
    pltpu.make_async_copy(zeros_hbm.at[rows], acc.at[rows], zsem).wait()
    plsc.subcore_barrier()

    @pl.loop(0, K, step=NBUF)
    def _(j):
        for b in range(NBUF):
            pltpu.make_async_copy(g_hbm.at[src_v.at[j + b]], bufs[b],
                                  gsems[b]).wait()
            pltpu.async_copy(bufs[b], acc.at[dst_v.at[j + b]], ssems[b],
                             add=True)
        for b in range(NBUF):
            @pl.when(j + NBUF + b < K)
            def _(b=b):
                pltpu.make_async_copy(bufs[b], acc.at[dst_v.at[j + b]],
                                      ssems[b]).wait()
                pltpu.async_copy(g_hbm.at[src_v.at[j + NBUF + b]], bufs[b],
                                 gsems[b])

    for b in range(NBUF):
        pltpu.make_async_copy(bufs[b], acc.at[dst_v.at[b]], ssems[b]).wait()
    plsc.subcore_barrier()

    @pl.when(cid == 0)
    def _():
        pltpu.sync_copy(acc.at[pl.ds(sid * GZ, GZ)], out_a.at[pl.ds(sid * GZ, GZ)])

    @pl.when(cid == 1)
    def _():
        pltpu.sync_copy(acc.at[pl.ds(sid * GZ, GZ)], out_b.at[pl.ds(sid * GZ, GZ)])


def _sc_aggregate(dp, g, src_r, dst_r, zeros):
    mesh = plsc.VectorSubcoreMesh(core_axis_name="c", subcore_axis_name="s")
    part = jax.ShapeDtypeStruct((NACC, dp), jnp.float32)
    kern = pl.kernel(
        functools.partial(_sc_agg_body, dp),
        out_type=(part, part),
        mesh=mesh,
        scratch_types=(
            [pltpu.VMEM((K, CH), jnp.int32),
             pltpu.VMEM((K, CH), jnp.int32)]
            + [pltpu.VMEM((CH, dp), jnp.float32) for _ in range(NBUF)]
            + [pltpu.VMEM_SHARED((NACC, dp), jnp.float32)]
            + [pltpu.SemaphoreType.DMA for _ in range(2 * NBUF + 1)]
        ),
        name=f"sc_segsum_d{dp}",
        compiler_params=pltpu.CompilerParams(use_tc_tiling_on_sc=False),
    )
    return kern(g, src_r, dst_r, zeros)


def _mm_aug_kernel(x_ref, wa_ref, wb_ref, oa_ref, ob_ref):
    x = x_ref[...]
    ya = jnp.dot(x, wa_ref[...], preferred_element_type=jnp.float32)
    col = lax.broadcasted_iota(jnp.int32, (BR, DPA), 1)
    oa_ref[...] = ya + jnp.where(col == DH, 1.0, 0.0).astype(jnp.float32)
    ob_ref[...] = jnp.dot(x, wb_ref[...], preferred_element_type=jnp.float32)


def _stage2_kernel(saa_ref, sba_ref, sab_ref, sbb_ref, g1a_ref, g1b_ref,
                   b1_ref, w2_ref, g2_ref, inv_ref):
    ma = saa_ref[...] + sba_ref[...]
    mb = sab_ref[...] + sbb_ref[...]
    deg = ma[:, DH:DH + 1]
    inv = 1.0 / (deg + 1.0)
    agg = jnp.concatenate(
        [ma[:, :DH] + g1a_ref[:, :DH], mb + g1b_ref[...]], axis=1)
    h = jnp.maximum(agg * inv + b1_ref[...], 0.0)
    g2_ref[...] = jnp.dot(h, w2_ref[...], preferred_element_type=jnp.float32)
    inv_ref[...] = inv


def _stage3_kernel(sa_ref, sb_ref, g2_ref, inv_ref, b2_ref, o_ref):
    o_ref[...] = ((sa_ref[...] + sb_ref[...] + g2_ref[...]) * inv_ref[...]
                  + b2_ref[...])


def kernel(features, edge_index, W1, b1, W2, b2):
    src = edge_index[0]
    dst = edge_index[1]
    pad = E_PAD - E
    src_r = jnp.concatenate([src, jnp.zeros((pad,), jnp.int32)]).reshape(NW, K, CH)
    dst_pad = N + jnp.arange(pad, dtype=jnp.int32) % (NACC - N)
    dst_r = jnp.concatenate([dst, dst_pad]).reshape(NW, K, CH)
    zeros_a = jnp.zeros((NACC, DPA), jnp.float32)
    zeros_b = jnp.zeros((NACC, DPB), jnp.float32)
    w1a = jnp.pad(W1[:, :DH], ((0, 0), (0, DPA - DH)))
    w1b = W1[:, DH:]
    b1r = b1.reshape(1, D_HID)
    b2r = b2.reshape(1, N_CLASSES)

    grid = N // BR

    g1a, g1b = pl.pallas_call(
        _mm_aug_kernel,
        grid=(grid,),
        in_specs=[
            pl.BlockSpec((BR, D_IN), lambda i: (i, 0)),
            pl.BlockSpec((D_IN, DPA), lambda i: (0, 0)),
            pl.BlockSpec((D_IN, DPB), lambda i: (0, 0)),
        ],
        out_specs=[
            pl.BlockSpec((BR, DPA), lambda i: (i, 0)),
            pl.BlockSpec((BR, DPB), lambda i: (i, 0)),
        ],
        out_shape=[
            jax.ShapeDtypeStruct((N, DPA), jnp.float32),
            jax.ShapeDtypeStruct((N, DPB), jnp.float32),
        ],
    )(features, w1a, w1b)

    saa, sba = _sc_aggregate(DPA, g1a, src_r, dst_r, zeros_a)
    sab, sbb = _sc_aggregate(DPB, g1b, src_r, dst_r, zeros_b)

    g2, inv = pl.pallas_call(
        _stage2_kernel,
        grid=(grid,),
        in_specs=[
            pl.BlockSpec((BR, DPA), lambda i: (i, 0)),
            pl.BlockSpec((BR, DPA), lambda i: (i, 0)),
            pl.BlockSpec((BR, DPB), lambda i: (i, 0)),
            pl.BlockSpec((BR, DPB), lambda i: (i, 0)),
            pl.BlockSpec((BR, DPA), lambda i: (i, 0)),
            pl.BlockSpec((BR, DPB), lambda i: (i, 0)),
            pl.BlockSpec((1, D_HID), lambda i: (0, 0)),
            pl.BlockSpec((D_HID, N_CLASSES), lambda i: (0, 0)),
        ],
        out_specs=[
            pl.BlockSpec((BR, N_CLASSES), lambda i: (i, 0)),
            pl.BlockSpec((BR, 1), lambda i: (i, 0)),
        ],
        out_shape=[
            jax.ShapeDtypeStruct((N, N_CLASSES), jnp.float32),
            jax.ShapeDtypeStruct((N, 1), jnp.float32),
        ],
    )(saa, sba, sab, sbb, g1a, g1b, b1r, W2)

    sa2, sb2 = _sc_aggregate(DP2, g2, src_r, dst_r, zeros_b)

    out = pl.pallas_call(
        _stage3_kernel,
        grid=(grid,),
        in_specs=[
            pl.BlockSpec((BR, N_CLASSES), lambda i: (i, 0)),
            pl.BlockSpec((BR, N_CLASSES), lambda i: (i, 0)),
            pl.BlockSpec((BR, N_CLASSES), lambda i: (i, 0)),
            pl.BlockSpec((BR, 1), lambda i: (i, 0)),
            pl.BlockSpec((1, N_CLASSES), lambda i: (0, 0)),
        ],
        out_specs=pl.BlockSpec((BR, N_CLASSES), lambda i: (i, 0)),
        out_shape=jax.ShapeDtypeStruct((N, N_CLASSES), jnp.float32),
    )(sa2, sb2, g2, inv, b2r)

    return out


# speedup vs baseline: 10.7315x; 2.9273x over previous
import functools

import jax
import jax.numpy as jnp
from jax import lax
from jax.experimental import pallas as pl
from jax.experimental.pallas import tpu as pltpu
from jax.experimental.pallas import tpu_sc as plsc

N = 10000
E = 320000
D_IN = 128
D_HID = 128
N_CLASSES = 64
DH = 64

NC = 2
NS = 16
NW = NC * NS
CH = 128
K = 80
E_PAD = NW * K * CH
NACC = 10112
GZ = NACC // NS

DPA = 80
DPB = 64
DP2 = N_CLASSES

BR = 400


NBUF = 4


def _sc_agg_body(dp, g_hbm, src_hbm, dst_hbm, zeros_hbm, out_a, out_b,
                 src_v, dst_v, b0, b1, b2, b3, acc,
                 zsem, g0, g1, g2, g3, s0, s1, s2, s3):
    bufs = (b0, b1, b2, b3)
    gsems = (g0, g1, g2, g3)
    ssems = (s0, s1, s2, s3)
    cid = lax.axis_index("c")
    sid = lax.axis_index("s")
    wid = sid * NC + cid
    rows = pl.ds(sid * GZ, GZ)

    pltpu.async_copy(zeros_hbm.at[rows], acc.at[rows], zsem)
    pltpu.sync_copy(src_hbm.at[wid], src_v)
    pltpu.sync_copy(dst_hbm.at[wid], dst_v)
    for b in range(NBUF):
        pltpu.async_copy(g_hbm.at[src_v.at[b]], bufs[b], gsems[b])
    pltpu.make_async_copy(zeros_hbm.at[rows], acc.at[rows], zsem).wait()
    plsc.subcore_barrier()

    @pl.loop(0, K, step=NBUF)
    def _(j):
        for b in range(NBUF):
            pltpu.make_async_copy(g_hbm.at[src_v.at[j + b]], bufs[b],
                                  gsems[b]).wait()
            pltpu.async_copy(bufs[b], acc.at[dst_v.at[j + b]], ssems[b],
                             add=True)
        for b in range(NBUF):
            @pl.when(j + NBUF + b < K)
            def _(b=b):
                pltpu.make_async_copy(bufs[b], acc.at[dst_v.at[j + b]],
                                      ssems[b]).wait()
                pltpu.async_copy(g_hbm.at[src_v.at[j + NBUF + b]], bufs[b],
                                 gsems[b])

    for b in range(NBUF):
        pltpu.make_async_copy(bufs[b], acc.at[dst_v.at[b]], ssems[b]).wait()
    plsc.subcore_barrier()

    @pl.when(cid == 0)
    def _():
        pltpu.sync_copy(acc.at[pl.ds(sid * GZ, GZ)], out_a.at[pl.ds(sid * GZ, GZ)])

    @pl.when(cid == 1)
    def _():
        pltpu.sync_copy(acc.at[pl.ds(sid * GZ, GZ)], out_b.at[pl.ds(sid * GZ, GZ)])


def _sc_aggregate(dp, g, src_r, dst_r, zeros):
    mesh = plsc.VectorSubcoreMesh(core_axis_name="c", subcore_axis_name="s")
    part = jax.ShapeDtypeStruct((NACC, dp), jnp.float32)
    kern = pl.kernel(
        functools.partial(_sc_agg_body, dp),
        out_type=(part, part),
        mesh=mesh,
        scratch_types=(
            [pltpu.VMEM((K, CH), jnp.int32),
             pltpu.VMEM((K, CH), jnp.int32)]
            + [pltpu.VMEM((CH, dp), jnp.float32) for _ in range(NBUF)]
            + [pltpu.VMEM_SHARED((NACC, dp), jnp.float32)]
            + [pltpu.SemaphoreType.DMA for _ in range(2 * NBUF + 1)]
        ),
        name=f"sc_segsum_d{dp}",
        compiler_params=pltpu.CompilerParams(use_tc_tiling_on_sc=False),
    )
    return kern(g, src_r, dst_r, zeros)


def _mm_aug_kernel(x_ref, wa_ref, wb_ref, oa_ref, ob_ref):
    x = x_ref[...]
    ya = jnp.dot(x, wa_ref[...], preferred_element_type=jnp.float32)
    col = lax.broadcasted_iota(jnp.int32, (BR, DPA), 1)
    oa_ref[...] = ya + jnp.where(col == DH, 1.0, 0.0).astype(jnp.float32)
    ob_ref[...] = jnp.dot(x, wb_ref[...], preferred_element_type=jnp.float32)


def _stage2_kernel(saa_ref, sba_ref, sab_ref, sbb_ref, g1a_ref, g1b_ref,
                   b1_ref, w2_ref, g2_ref, inv_ref):
    ma = saa_ref[...] + sba_ref[...]
    mb = sab_ref[...] + sbb_ref[...]
    deg = ma[:, DH:DH + 1]
    inv = 1.0 / (deg + 1.0)
    agg = jnp.concatenate(
        [ma[:, :DH] + g1a_ref[:, :DH], mb + g1b_ref[...]], axis=1)
    h = jnp.maximum(agg * inv + b1_ref[...], 0.0)
    g2_ref[...] = jnp.dot(h, w2_ref[...], preferred_element_type=jnp.float32)
    inv_ref[...] = inv


def _stage3_kernel(sa_ref, sb_ref, g2_ref, inv_ref, b2_ref, o_ref):
    o_ref[...] = ((sa_ref[...] + sb_ref[...] + g2_ref[...]) * inv_ref[...]
                  + b2_ref[...])


def kernel(features, edge_index, W1, b1, W2, b2):
    src = edge_index[0]
    dst = edge_index[1]
    pad = E_PAD - E
    per_tile_pad = pad // NW
    src_pad = (jnp.arange(pad, dtype=jnp.int32) % N).reshape(NW, per_tile_pad)
    dst_pad = (N + jnp.arange(pad, dtype=jnp.int32) % (NACC - N)).reshape(
        NW, per_tile_pad).astype(jnp.int32)
    src_r = jnp.concatenate(
        [src.reshape(NW, E // NW), src_pad], axis=1).reshape(NW, K, CH)
    dst_r = jnp.concatenate(
        [dst.reshape(NW, E // NW), dst_pad], axis=1).reshape(NW, K, CH)
    zeros_a = jnp.zeros((NACC, DPA), jnp.float32)
    zeros_b = jnp.zeros((NACC, DPB), jnp.float32)
    w1a = jnp.pad(W1[:, :DH], ((0, 0), (0, DPA - DH)))
    w1b = W1[:, DH:]
    b1r = b1.reshape(1, D_HID)
    b2r = b2.reshape(1, N_CLASSES)

    grid = N // BR

    g1a, g1b = pl.pallas_call(
        _mm_aug_kernel,
        grid=(grid,),
        in_specs=[
            pl.BlockSpec((BR, D_IN), lambda i: (i, 0)),
            pl.BlockSpec((D_IN, DPA), lambda i: (0, 0)),
            pl.BlockSpec((D_IN, DPB), lambda i: (0, 0)),
        ],
        out_specs=[
            pl.BlockSpec((BR, DPA), lambda i: (i, 0)),
            pl.BlockSpec((BR, DPB), lambda i: (i, 0)),
        ],
        out_shape=[
            jax.ShapeDtypeStruct((N, DPA), jnp.float32),
            jax.ShapeDtypeStruct((N, DPB), jnp.float32),
        ],
    )(features, w1a, w1b)

    saa, sba = _sc_aggregate(DPA, g1a, src_r, dst_r, zeros_a)
    sab, sbb = _sc_aggregate(DPB, g1b, src_r, dst_r, zeros_b)

    g2, inv = pl.pallas_call(
        _stage2_kernel,
        grid=(grid,),
        in_specs=[
            pl.BlockSpec((BR, DPA), lambda i: (i, 0)),
            pl.BlockSpec((BR, DPA), lambda i: (i, 0)),
            pl.BlockSpec((BR, DPB), lambda i: (i, 0)),
            pl.BlockSpec((BR, DPB), lambda i: (i, 0)),
            pl.BlockSpec((BR, DPA), lambda i: (i, 0)),
            pl.BlockSpec((BR, DPB), lambda i: (i, 0)),
            pl.BlockSpec((1, D_HID), lambda i: (0, 0)),
            pl.BlockSpec((D_HID, N_CLASSES), lambda i: (0, 0)),
        ],
        out_specs=[
            pl.BlockSpec((BR, N_CLASSES), lambda i: (i, 0)),
            pl.BlockSpec((BR, 1), lambda i: (i, 0)),
        ],
        out_shape=[
            jax.ShapeDtypeStruct((N, N_CLASSES), jnp.float32),
            jax.ShapeDtypeStruct((N, 1), jnp.float32),
        ],
    )(saa, sba, sab, sbb, g1a, g1b, b1r, W2)

    sa2, sb2 = _sc_aggregate(DP2, g2, src_r, dst_r, zeros_b)

    out = pl.pallas_call(
        _stage3_kernel,
        grid=(grid,),
        in_specs=[
            pl.BlockSpec((BR, N_CLASSES), lambda i: (i, 0)),
            pl.BlockSpec((BR, N_CLASSES), lambda i: (i, 0)),
            pl.BlockSpec((BR, N_CLASSES), lambda i: (i, 0)),
            pl.BlockSpec((BR, 1), lambda i: (i, 0)),
            pl.BlockSpec((1, N_CLASSES), lambda i: (0, 0)),
        ],
        out_specs=pl.BlockSpec((BR, N_CLASSES), lambda i: (i, 0)),
        out_shape=jax.ShapeDtypeStruct((N, N_CLASSES), jnp.float32),
    )(sa2, sb2, g2, inv, b2r)

    return out


# speedup vs baseline: 11.6255x; 1.0833x over previous
import functools

import jax
import jax.numpy as jnp
from jax import lax
from jax.experimental import pallas as pl
from jax.experimental.pallas import tpu as pltpu
from jax.experimental.pallas import tpu_sc as plsc

N = 10000
E = 320000
D_IN = 128
D_HID = 128
N_CLASSES = 64
DH = 64

NC = 2
NS = 16
NW = NC * NS
CH = 128
K = 80
E_PAD = NW * K * CH
NACC = 10112
GZ = NACC // NS

DPA = 80
DPB = 64
DP2 = N_CLASSES

BR = 2000


NBUF = 4


def _sc_agg_body(dp, g_hbm, src_hbm, dst_hbm, zeros_hbm, out_a, out_b,
                 src_v, dst_v, b0, b1, b2, b3, acc,
                 zsem, g0, g1, g2, g3, s0, s1, s2, s3):
    bufs = (b0, b1, b2, b3)
    gsems = (g0, g1, g2, g3)
    ssems = (s0, s1, s2, s3)
    cid = lax.axis_index("c")
    sid = lax.axis_index("s")
    wid = sid * NC + cid
    rows = pl.ds(sid * GZ, GZ)

    pltpu.async_copy(zeros_hbm.at[rows], acc.at[rows], zsem)
    pltpu.sync_copy(src_hbm.at[wid], src_v)
    pltpu.sync_copy(dst_hbm.at[wid], dst_v)
    for b in range(NBUF):
        pltpu.async_copy(g_hbm.at[src_v.at[b]], bufs[b], gsems[b])
    pltpu.make_async_copy(zeros_hbm.at[rows], acc.at[rows], zsem).wait()
    plsc.subcore_barrier()

    @pl.loop(0, K, step=NBUF)
    def _(j):
        for b in range(NBUF):
            pltpu.make_async_copy(g_hbm.at[src_v.at[j + b]], bufs[b],
                                  gsems[b]).wait()
            pltpu.async_copy(bufs[b], acc.at[dst_v.at[j + b]], ssems[b],
                             add=True)
        for b in range(NBUF):
            @pl.when(j + NBUF + b < K)
            def _(b=b):
                pltpu.make_async_copy(bufs[b], acc.at[dst_v.at[j + b]],
                                      ssems[b]).wait()
                pltpu.async_copy(g_hbm.at[src_v.at[j + NBUF + b]], bufs[b],
                                 gsems[b])

    for b in range(NBUF):
        pltpu.make_async_copy(bufs[b], acc.at[dst_v.at[b]], ssems[b]).wait()
    plsc.subcore_barrier()

    @pl.when(cid == 0)
    def _():
        pltpu.sync_copy(acc.at[pl.ds(sid * GZ, GZ)], out_a.at[pl.ds(sid * GZ, GZ)])

    @pl.when(cid == 1)
    def _():
        pltpu.sync_copy(acc.at[pl.ds(sid * GZ, GZ)], out_b.at[pl.ds(sid * GZ, GZ)])


def _sc_aggregate(dp, g, src_r, dst_r, zeros):
    mesh = plsc.VectorSubcoreMesh(core_axis_name="c", subcore_axis_name="s")
    part = jax.ShapeDtypeStruct((NACC, dp), jnp.float32)
    kern = pl.kernel(
        functools.partial(_sc_agg_body, dp),
        out_type=(part, part),
        mesh=mesh,
        scratch_types=(
            [pltpu.VMEM((K, CH), jnp.int32),
             pltpu.VMEM((K, CH), jnp.int32)]
            + [pltpu.VMEM((CH, dp), jnp.float32) for _ in range(NBUF)]
            + [pltpu.VMEM_SHARED((NACC, dp), jnp.float32)]
            + [pltpu.SemaphoreType.DMA for _ in range(2 * NBUF + 1)]
        ),
        name=f"sc_segsum_d{dp}",
        compiler_params=pltpu.CompilerParams(use_tc_tiling_on_sc=False),
    )
    return kern(g, src_r, dst_r, zeros)


def _mm_aug_kernel(x_ref, wa_ref, wb_ref, oa_ref, ob_ref):
    x = x_ref[...]
    ya = jnp.dot(x, wa_ref[...], preferred_element_type=jnp.float32,
                 precision=lax.Precision.HIGHEST)
    col = lax.broadcasted_iota(jnp.int32, (BR, DPA), 1)
    oa_ref[...] = ya + jnp.where(col == DH, 1.0, 0.0).astype(jnp.float32)
    ob_ref[...] = jnp.dot(x, wb_ref[...], preferred_element_type=jnp.float32,
                          precision=lax.Precision.HIGHEST)


def _stage2_kernel(saa_ref, sba_ref, sab_ref, sbb_ref, g1a_ref, g1b_ref,
                   b1_ref, w2_ref, g2_ref, inv_ref):
    ma = saa_ref[...] + sba_ref[...]
    mb = sab_ref[...] + sbb_ref[...]
    deg = ma[:, DH:DH + 1]
    inv = 1.0 / (deg + 1.0)
    agg = jnp.concatenate(
        [ma[:, :DH] + g1a_ref[:, :DH], mb + g1b_ref[...]], axis=1)
    h = jnp.maximum(agg * inv + b1_ref[...], 0.0)
    g2_ref[...] = jnp.dot(h, w2_ref[...], preferred_element_type=jnp.float32,
                          precision=lax.Precision.HIGHEST)
    inv_ref[...] = inv


def _stage3_kernel(sa_ref, sb_ref, g2_ref, inv_ref, b2_ref, o_ref):
    o_ref[...] = ((sa_ref[...] + sb_ref[...] + g2_ref[...]) * inv_ref[...]
                  + b2_ref[...])


def kernel(features, edge_index, W1, b1, W2, b2):
    src = edge_index[0]
    dst = edge_index[1]
    pad = E_PAD - E
    per_tile_pad = pad // NW
    src_pad = (jnp.arange(pad, dtype=jnp.int32) % N).reshape(NW, per_tile_pad)
    dst_pad = (N + jnp.arange(pad, dtype=jnp.int32) % (NACC - N)).reshape(
        NW, per_tile_pad).astype(jnp.int32)
    src_r = jnp.concatenate(
        [src.reshape(NW, E // NW), src_pad], axis=1).reshape(NW, K, CH)
    dst_r = jnp.concatenate(
        [dst.reshape(NW, E // NW), dst_pad], axis=1).reshape(NW, K, CH)
    zeros_a = jnp.zeros((NACC, DPA), jnp.float32)
    zeros_b = jnp.zeros((NACC, DPB), jnp.float32)
    w1a = jnp.pad(W1[:, :DH], ((0, 0), (0, DPA - DH)))
    w1b = W1[:, DH:]
    b1r = b1.reshape(1, D_HID)
    b2r = b2.reshape(1, N_CLASSES)

    grid = N // BR

    g1a, g1b = pl.pallas_call(
        _mm_aug_kernel,
        grid=(grid,),
        in_specs=[
            pl.BlockSpec((BR, D_IN), lambda i: (i, 0)),
            pl.BlockSpec((D_IN, DPA), lambda i: (0, 0)),
            pl.BlockSpec((D_IN, DPB), lambda i: (0, 0)),
        ],
        out_specs=[
            pl.BlockSpec((BR, DPA), lambda i: (i, 0)),
            pl.BlockSpec((BR, DPB), lambda i: (i, 0)),
        ],
        out_shape=[
            jax.ShapeDtypeStruct((N, DPA), jnp.float32),
            jax.ShapeDtypeStruct((N, DPB), jnp.float32),
        ],
    )(features, w1a, w1b)

    saa, sba = _sc_aggregate(DPA, g1a, src_r, dst_r, zeros_a)
    sab, sbb = _sc_aggregate(DPB, g1b, src_r, dst_r, zeros_b)

    g2, inv = pl.pallas_call(
        _stage2_kernel,
        grid=(grid,),
        in_specs=[
            pl.BlockSpec((BR, DPA), lambda i: (i, 0)),
            pl.BlockSpec((BR, DPA), lambda i: (i, 0)),
            pl.BlockSpec((BR, DPB), lambda i: (i, 0)),
            pl.BlockSpec((BR, DPB), lambda i: (i, 0)),
            pl.BlockSpec((BR, DPA), lambda i: (i, 0)),
            pl.BlockSpec((BR, DPB), lambda i: (i, 0)),
            pl.BlockSpec((1, D_HID), lambda i: (0, 0)),
            pl.BlockSpec((D_HID, N_CLASSES), lambda i: (0, 0)),
        ],
        out_specs=[
            pl.BlockSpec((BR, N_CLASSES), lambda i: (i, 0)),
            pl.BlockSpec((BR, 1), lambda i: (i, 0)),
        ],
        out_shape=[
            jax.ShapeDtypeStruct((N, N_CLASSES), jnp.float32),
            jax.ShapeDtypeStruct((N, 1), jnp.float32),
        ],
    )(saa, sba, sab, sbb, g1a, g1b, b1r, W2)

    sa2, sb2 = _sc_aggregate(DP2, g2, src_r, dst_r, zeros_b)

    out = pl.pallas_call(
        _stage3_kernel,
        grid=(grid,),
        in_specs=[
            pl.BlockSpec((BR, N_CLASSES), lambda i: (i, 0)),
            pl.BlockSpec((BR, N_CLASSES), lambda i: (i, 0)),
            pl.BlockSpec((BR, N_CLASSES), lambda i: (i, 0)),
            pl.BlockSpec((BR, 1), lambda i: (i, 0)),
            pl.BlockSpec((1, N_CLASSES), lambda i: (0, 0)),
        ],
        out_specs=pl.BlockSpec((BR, N_CLASSES), lambda i: (i, 0)),
        out_shape=jax.ShapeDtypeStruct((N, N_CLASSES), jnp.float32),
    )(sa2, sb2, g2, inv, b2r)

    return out
